# baseline (speedup 1.0000x reference)
import functools

import jax
import jax.numpy as jnp
from jax import lax
from jax.experimental import pallas as pl
from jax.experimental.pallas import tpu as pltpu
from jax.experimental.pallas import tpu_sc as plsc

B = 4096
S = 200
E = 64
H = 512
C = 128

NW = 32
BPW = B // NW
CH = 100
ROWS_PER_W = BPW * S // CH


def _pool_sc(x_r, table):
    mesh = plsc.VectorSubcoreMesh(core_axis_name="c", subcore_axis_name="s")

    @functools.partial(
        pl.kernel,
        out_type=jax.ShapeDtypeStruct((B, E), jnp.float32),
        mesh=mesh,
        compiler_params=pltpu.CompilerParams(use_tc_tiling_on_sc=False),
        scratch_types=[
            pltpu.VMEM((ROWS_PER_W, CH), jnp.int32),
            pltpu.VMEM((2, 2, CH, E), jnp.float32),
            pltpu.VMEM((BPW, E), jnp.float32),
            pltpu.SemaphoreType.DMA,
            pltpu.SemaphoreType.DMA,
        ],
    )
    def k(x_hbm, table_hbm, out_hbm, idx_v, rows_v, acc_v, sem0, sem1):
        nc = 2
        wid = lax.axis_index("s") * nc + lax.axis_index("c")
        pltpu.sync_copy(x_hbm.at[wid], idx_v)
        sems = [sem0, sem1]

        def issue(i, b):
            for h in range(2):
                pltpu.make_async_copy(
                    table_hbm.at[idx_v.at[2 * i + h]],
                    rows_v.at[b, h], sems[b]).start()

        def wait_g(i, b):
            for h in range(2):
                pltpu.make_async_copy(
                    table_hbm.at[idx_v.at[2 * i + h]],
                    rows_v.at[b, h], sems[b]).wait()

        def reduce_into(i, b):
            def body(r, carry):
                out = []
                for c in range(4):
                    v0 = rows_v[b, 0, r, pl.ds(16 * c, 16)]
                    v1 = rows_v[b, 1, r, pl.ds(16 * c, 16)]
                    out.append(carry[c] + v0 + v1)
                return tuple(out)
            init = tuple(jnp.zeros((16,), jnp.float32) for _ in range(4))
            acc = lax.fori_loop(0, CH, body, init)
            for c in range(4):
                acc_v[i, pl.ds(16 * c, 16)] = acc[c]

        issue(0, 0)
        issue(1, 1)

        def outer(g, carry):
            for bb in range(2):
                i = 2 * g + bb
                wait_g(i, bb)

                @pl.when(i + 2 < BPW)
                def _():
                    issue(i + 2, bb)

                reduce_into(i, bb)
            return carry

        lax.fori_loop(0, BPW // 2, outer, 0)
        pltpu.sync_copy(acc_v, out_hbm.at[pl.ds(wid * BPW, BPW)])

    return k(x_r, table)


def _mlp_tc(pooled, W1, b1, W2, b2):
    BLK = 512

    def body(p_ref, w1_ref, b1_ref, w2_ref, b2_ref, o_ref):
        p = p_ref[...] * (1.0 / S)
        h = jnp.dot(p, w1_ref[...], preferred_element_type=jnp.float32)
        h = jnp.maximum(h + b1_ref[...], 0.0)
        o_ref[...] = jnp.dot(h, w2_ref[...],
                             preferred_element_type=jnp.float32) + b2_ref[...]

    return pl.pallas_call(
        body,
        grid=(B // BLK,),
        in_specs=[
            pl.BlockSpec((BLK, E), lambda i: (i, 0)),
            pl.BlockSpec((E, H), lambda i: (0, 0)),
            pl.BlockSpec((1, H), lambda i: (0, 0)),
            pl.BlockSpec((H, C), lambda i: (0, 0)),
            pl.BlockSpec((1, C), lambda i: (0, 0)),
        ],
        out_specs=pl.BlockSpec((BLK, C), lambda i: (i, 0)),
        out_shape=jax.ShapeDtypeStruct((B, C), jnp.float32),
    )(pooled, W1, b1.reshape(1, H), W2, b2.reshape(1, C))


def kernel(x, table, W1, b1, W2, b2):
    x_r = x.astype(jnp.int32).reshape(NW, ROWS_PER_W, CH)
    pooled = _pool_sc(x_r, table)
    return _mlp_tc(pooled, W1, b1, W2, b2)
